# baseline (speedup 1.0000x reference)
import jax
import jax.numpy as jnp
from jax.experimental import pallas as pl
from jax.experimental.pallas import tpu as pltpu

_EPS = 1e-05


def _stats_kernel(x_ref, m_ref, s_ref, ss_ref, cnt_ref):
    xb = x_ref[0]
    vf = (m_ref[0, 0] > 0.5).astype(jnp.float32)
    xm = xb * vf[None]
    s1 = jnp.sum(xm, axis=1)
    s_ref[0] = jnp.sum(s1, axis=1, keepdims=True)
    sq = (xb * xb) * vf[None]
    t1 = jnp.sum(sq, axis=1)
    ss_ref[0] = jnp.sum(t1, axis=1, keepdims=True)
    c1 = jnp.sum(vf, axis=0, keepdims=True)
    cnt_ref[0] = jnp.sum(c1, axis=1, keepdims=True)


def _fold_kernel(s_ref, ss_ref, cnt_ref, w_ref, b_ref, scale_ref, shift_ref):
    s = s_ref[:, :, 0]
    ss = ss_ref[:, :, 0]
    cnt = cnt_ref[:, :, 0]
    safe = jnp.maximum(cnt, 1.0)
    mean = jnp.where(cnt > 0, s / safe, 0.0)
    sq = ss - 2.0 * mean * s + mean * mean * cnt
    var = jnp.where(cnt > 1, sq / safe, 1.0)
    b_count = s_ref.shape[0]
    fm = jnp.sum(mean, axis=0) / b_count
    fv = jnp.sum(var, axis=0) / b_count
    scale = w_ref[0] * jax.lax.rsqrt(fv + _EPS)
    shift = b_ref[0] - fm * scale
    scale_ref[...] = scale[:, None]
    shift_ref[...] = shift[:, None]


def _norm_kernel(scale_ref, shift_ref, x_ref, m_ref, o_ref):
    sc = scale_ref[...]
    sh = shift_ref[...]
    xb = x_ref[0]
    valid = m_ref[0, 0] > 0.5
    xn = xb * sc[:, :, None] + sh[:, :, None]
    o_ref[0] = jnp.where(valid[None], xn, xb)


def kernel(x, mask, weight, bias):
    B, C, H, W = x.shape

    NC = 2
    Cb = C // NC
    s, ss, cnt = pl.pallas_call(
        _stats_kernel,
        grid=(B, NC),
        in_specs=[
            pl.BlockSpec((1, Cb, H, W), lambda b, c: (b, c, 0, 0)),
            pl.BlockSpec((1, 1, H, W), lambda b, c: (b, 0, 0, 0)),
        ],
        out_specs=[
            pl.BlockSpec((1, Cb, 1), lambda b, c: (b, c, 0)),
            pl.BlockSpec((1, Cb, 1), lambda b, c: (b, c, 0)),
            pl.BlockSpec((1, 1, 1), lambda b, c: (b, 0, 0)),
        ],
        out_shape=[
            jax.ShapeDtypeStruct((B, C, 1), jnp.float32),
            jax.ShapeDtypeStruct((B, C, 1), jnp.float32),
            jax.ShapeDtypeStruct((B, 1, 1), jnp.float32),
        ],
        compiler_params=pltpu.CompilerParams(
            dimension_semantics=("parallel", "arbitrary"),
            vmem_limit_bytes=52 * 1024 * 1024,
        ),
        name="masked_bn_stats",
    )(x, mask)

    w2 = weight.reshape(1, C)
    b2 = bias.reshape(1, C)

    scale, shift = pl.pallas_call(
        _fold_kernel,
        grid=(1,),
        in_specs=[
            pl.BlockSpec((B, C, 1), lambda i: (0, 0, 0)),
            pl.BlockSpec((B, C, 1), lambda i: (0, 0, 0)),
            pl.BlockSpec((B, 1, 1), lambda i: (0, 0, 0)),
            pl.BlockSpec((1, C), lambda i: (0, 0)),
            pl.BlockSpec((1, C), lambda i: (0, 0)),
        ],
        out_specs=[
            pl.BlockSpec((C, 1), lambda i: (0, 0)),
            pl.BlockSpec((C, 1), lambda i: (0, 0)),
        ],
        out_shape=[
            jax.ShapeDtypeStruct((C, 1), jnp.float32),
            jax.ShapeDtypeStruct((C, 1), jnp.float32),
        ],
        name="masked_bn_fold",
    )(s, ss, cnt, w2, b2)

    NJ = 2
    Hb = H // NJ
    out = pl.pallas_call(
        _norm_kernel,
        grid=(B, NJ),
        in_specs=[
            pl.BlockSpec((C, 1), lambda b, j: (0, 0)),
            pl.BlockSpec((C, 1), lambda b, j: (0, 0)),
            pl.BlockSpec((1, C, Hb, W), lambda b, j: (b, 0, j, 0)),
            pl.BlockSpec((1, 1, Hb, W), lambda b, j: (b, 0, j, 0)),
        ],
        out_specs=pl.BlockSpec((1, C, Hb, W), lambda b, j: (b, 0, j, 0)),
        out_shape=jax.ShapeDtypeStruct((B, C, H, W), jnp.float32),
        compiler_params=pltpu.CompilerParams(
            dimension_semantics=("parallel", "arbitrary"),
            vmem_limit_bytes=52 * 1024 * 1024,
        ),
        name="masked_bn_norm",
    )(scale, shift, x, mask)

    return out


# speedup vs baseline: 1.0318x; 1.0318x over previous
import jax
import jax.numpy as jnp
from jax.experimental import pallas as pl
from jax.experimental.pallas import tpu as pltpu

_EPS = 1e-05


def _stats_kernel(x_ref, m_ref, s_ref, ss_ref, cnt_ref):
    xb = x_ref[0]
    vf = (m_ref[0, 0] > 0.5).astype(jnp.float32)
    xm = xb * vf[None]
    s1 = jnp.sum(xm, axis=1)
    s_ref[0] = jnp.sum(s1, axis=1, keepdims=True)
    sq = (xb * xb) * vf[None]
    t1 = jnp.sum(sq, axis=1)
    ss_ref[0] = jnp.sum(t1, axis=1, keepdims=True)
    c1 = jnp.sum(vf, axis=0, keepdims=True)
    cnt_ref[0] = jnp.sum(c1, axis=1, keepdims=True)


def _fold_kernel(s_ref, ss_ref, cnt_ref, w_ref, b_ref, scale_ref, shift_ref):
    s = s_ref[:, :, 0]
    ss = ss_ref[:, :, 0]
    cnt = cnt_ref[:, :, 0]
    safe = jnp.maximum(cnt, 1.0)
    mean = jnp.where(cnt > 0, s / safe, 0.0)
    sq = ss - 2.0 * mean * s + mean * mean * cnt
    var = jnp.where(cnt > 1, sq / safe, 1.0)
    b_count = s_ref.shape[0]
    fm = jnp.sum(mean, axis=0) / b_count
    fv = jnp.sum(var, axis=0) / b_count
    scale = w_ref[0] * jax.lax.rsqrt(fv + _EPS)
    shift = b_ref[0] - fm * scale
    scale_ref[...] = scale[:, None]
    shift_ref[...] = shift[:, None]


def _norm_kernel(scale_ref, shift_ref, x_ref, m_ref, o_ref):
    sc = scale_ref[...]
    sh = shift_ref[...]
    xb = x_ref[0]
    valid = m_ref[0, 0] > 0.5
    xn = xb * sc[:, :, None] + sh[:, :, None]
    o_ref[0] = jnp.where(valid[None], xn, xb)


def kernel(x, mask, weight, bias):
    B, C, H, W = x.shape

    s, ss, cnt = pl.pallas_call(
        _stats_kernel,
        grid=(B,),
        in_specs=[
            pl.BlockSpec((1, C, H, W), lambda b: (b, 0, 0, 0)),
            pl.BlockSpec((1, 1, H, W), lambda b: (b, 0, 0, 0)),
        ],
        out_specs=[
            pl.BlockSpec((1, C, 1), lambda b: (b, 0, 0)),
            pl.BlockSpec((1, C, 1), lambda b: (b, 0, 0)),
            pl.BlockSpec((1, 1, 1), lambda b: (b, 0, 0)),
        ],
        out_shape=[
            jax.ShapeDtypeStruct((B, C, 1), jnp.float32),
            jax.ShapeDtypeStruct((B, C, 1), jnp.float32),
            jax.ShapeDtypeStruct((B, 1, 1), jnp.float32),
        ],
        compiler_params=pltpu.CompilerParams(
            dimension_semantics=("parallel",),
            vmem_limit_bytes=52 * 1024 * 1024,
        ),
        name="masked_bn_stats",
    )(x, mask)

    w2 = weight.reshape(1, C)
    b2 = bias.reshape(1, C)

    scale, shift = pl.pallas_call(
        _fold_kernel,
        grid=(1,),
        in_specs=[
            pl.BlockSpec((B, C, 1), lambda i: (0, 0, 0)),
            pl.BlockSpec((B, C, 1), lambda i: (0, 0, 0)),
            pl.BlockSpec((B, 1, 1), lambda i: (0, 0, 0)),
            pl.BlockSpec((1, C), lambda i: (0, 0)),
            pl.BlockSpec((1, C), lambda i: (0, 0)),
        ],
        out_specs=[
            pl.BlockSpec((C, 1), lambda i: (0, 0)),
            pl.BlockSpec((C, 1), lambda i: (0, 0)),
        ],
        out_shape=[
            jax.ShapeDtypeStruct((C, 1), jnp.float32),
            jax.ShapeDtypeStruct((C, 1), jnp.float32),
        ],
        name="masked_bn_fold",
    )(s, ss, cnt, w2, b2)

    NJ = 2
    Hb = H // NJ
    out = pl.pallas_call(
        _norm_kernel,
        grid=(B, NJ),
        in_specs=[
            pl.BlockSpec((C, 1), lambda b, j: (0, 0)),
            pl.BlockSpec((C, 1), lambda b, j: (0, 0)),
            pl.BlockSpec((1, C, Hb, W), lambda b, j: (b, 0, j, 0)),
            pl.BlockSpec((1, 1, Hb, W), lambda b, j: (b, 0, j, 0)),
        ],
        out_specs=pl.BlockSpec((1, C, Hb, W), lambda b, j: (b, 0, j, 0)),
        out_shape=jax.ShapeDtypeStruct((B, C, H, W), jnp.float32),
        compiler_params=pltpu.CompilerParams(
            dimension_semantics=("parallel", "arbitrary"),
            vmem_limit_bytes=52 * 1024 * 1024,
        ),
        name="masked_bn_norm",
    )(scale, shift, x, mask)

    return out


# speedup vs baseline: 1.0874x; 1.0539x over previous
import jax
import jax.numpy as jnp
from jax.experimental import pallas as pl
from jax.experimental.pallas import tpu as pltpu

_EPS = 1e-05


def _stats_kernel(x_ref, m_ref, s_ref, ss_ref, cnt_ref):
    C, H, W = x_ref.shape[1:]
    vf = (m_ref[0, 0] > 0.5).astype(jnp.float32)
    CG = 4
    HC = 16
    for c0 in range(0, C, CG):
        acc_s = jnp.zeros((CG, HC, W), jnp.float32)
        acc_q = jnp.zeros((CG, HC, W), jnp.float32)
        for h0 in range(0, H, HC):
            xc = x_ref[0, c0:c0 + CG, h0:h0 + HC, :]
            xm = xc * vf[h0:h0 + HC][None]
            acc_s = acc_s + xm
            acc_q = acc_q + xm * xc
        s_cg = jnp.sum(jnp.sum(acc_s, axis=1), axis=1, keepdims=True)
        q_cg = jnp.sum(jnp.sum(acc_q, axis=1), axis=1, keepdims=True)
        s_ref[0, c0:c0 + CG] = s_cg
        ss_ref[0, c0:c0 + CG] = q_cg
    c1 = jnp.sum(vf, axis=0, keepdims=True)
    cnt_ref[0] = jnp.sum(c1, axis=1, keepdims=True)


def _fold_kernel(s_ref, ss_ref, cnt_ref, w_ref, b_ref, scale_ref, shift_ref):
    s = s_ref[:, :, 0]
    ss = ss_ref[:, :, 0]
    cnt = cnt_ref[:, :, 0]
    safe = jnp.maximum(cnt, 1.0)
    mean = jnp.where(cnt > 0, s / safe, 0.0)
    sq = ss - 2.0 * mean * s + mean * mean * cnt
    var = jnp.where(cnt > 1, sq / safe, 1.0)
    b_count = s_ref.shape[0]
    fm = jnp.sum(mean, axis=0) / b_count
    fv = jnp.sum(var, axis=0) / b_count
    scale = w_ref[0] * jax.lax.rsqrt(fv + _EPS)
    shift = b_ref[0] - fm * scale
    scale_ref[...] = scale[:, None]
    shift_ref[...] = shift[:, None]


def _norm_kernel(scale_ref, shift_ref, x_ref, m_ref, o_ref):
    sc = scale_ref[...]
    sh = shift_ref[...]
    xb = x_ref[0]
    valid = m_ref[0, 0] > 0.5
    xn = xb * sc[:, :, None] + sh[:, :, None]
    o_ref[0] = jnp.where(valid[None], xn, xb)


def kernel(x, mask, weight, bias):
    B, C, H, W = x.shape

    s, ss, cnt = pl.pallas_call(
        _stats_kernel,
        grid=(B,),
        in_specs=[
            pl.BlockSpec((1, C, H, W), lambda b: (b, 0, 0, 0)),
            pl.BlockSpec((1, 1, H, W), lambda b: (b, 0, 0, 0)),
        ],
        out_specs=[
            pl.BlockSpec((1, C, 1), lambda b: (b, 0, 0)),
            pl.BlockSpec((1, C, 1), lambda b: (b, 0, 0)),
            pl.BlockSpec((1, 1, 1), lambda b: (b, 0, 0)),
        ],
        out_shape=[
            jax.ShapeDtypeStruct((B, C, 1), jnp.float32),
            jax.ShapeDtypeStruct((B, C, 1), jnp.float32),
            jax.ShapeDtypeStruct((B, 1, 1), jnp.float32),
        ],
        compiler_params=pltpu.CompilerParams(
            dimension_semantics=("parallel",),
            vmem_limit_bytes=52 * 1024 * 1024,
        ),
        name="masked_bn_stats",
    )(x, mask)

    w2 = weight.reshape(1, C)
    b2 = bias.reshape(1, C)

    scale, shift = pl.pallas_call(
        _fold_kernel,
        grid=(1,),
        in_specs=[
            pl.BlockSpec((B, C, 1), lambda i: (0, 0, 0)),
            pl.BlockSpec((B, C, 1), lambda i: (0, 0, 0)),
            pl.BlockSpec((B, 1, 1), lambda i: (0, 0, 0)),
            pl.BlockSpec((1, C), lambda i: (0, 0)),
            pl.BlockSpec((1, C), lambda i: (0, 0)),
        ],
        out_specs=[
            pl.BlockSpec((C, 1), lambda i: (0, 0)),
            pl.BlockSpec((C, 1), lambda i: (0, 0)),
        ],
        out_shape=[
            jax.ShapeDtypeStruct((C, 1), jnp.float32),
            jax.ShapeDtypeStruct((C, 1), jnp.float32),
        ],
        name="masked_bn_fold",
    )(s, ss, cnt, w2, b2)

    NJ = 2
    Hb = H // NJ
    out = pl.pallas_call(
        _norm_kernel,
        grid=(B, NJ),
        in_specs=[
            pl.BlockSpec((C, 1), lambda b, j: (0, 0)),
            pl.BlockSpec((C, 1), lambda b, j: (0, 0)),
            pl.BlockSpec((1, C, Hb, W), lambda b, j: (b, 0, j, 0)),
            pl.BlockSpec((1, 1, Hb, W), lambda b, j: (b, 0, j, 0)),
        ],
        out_specs=pl.BlockSpec((1, C, Hb, W), lambda b, j: (b, 0, j, 0)),
        out_shape=jax.ShapeDtypeStruct((B, C, H, W), jnp.float32),
        compiler_params=pltpu.CompilerParams(
            dimension_semantics=("parallel", "arbitrary"),
            vmem_limit_bytes=52 * 1024 * 1024,
        ),
        name="masked_bn_norm",
    )(scale, shift, x, mask)

    return out


# speedup vs baseline: 1.0959x; 1.0078x over previous
import jax
import jax.numpy as jnp
from jax.experimental import pallas as pl
from jax.experimental.pallas import tpu as pltpu

_EPS = 1e-05


def _stats_kernel(x_ref, m_ref, s_ref, ss_ref, cnt_ref):
    C, H, W = x_ref.shape[1:]
    vf = (m_ref[0, 0] > 0.5).astype(jnp.float32)
    CG = 4
    HC = 16
    for c0 in range(0, C, CG):
        acc_s = jnp.zeros((CG, HC, W), jnp.float32)
        acc_q = jnp.zeros((CG, HC, W), jnp.float32)
        for h0 in range(0, H, HC):
            xc = x_ref[0, c0:c0 + CG, h0:h0 + HC, :]
            xm = xc * vf[h0:h0 + HC][None]
            acc_s = acc_s + xm
            acc_q = acc_q + xm * xc
        s_cg = jnp.sum(jnp.sum(acc_s, axis=1), axis=1, keepdims=True)
        q_cg = jnp.sum(jnp.sum(acc_q, axis=1), axis=1, keepdims=True)
        s_ref[0, c0:c0 + CG] = s_cg
        ss_ref[0, c0:c0 + CG] = q_cg
    c1 = jnp.sum(vf, axis=0, keepdims=True)
    cnt_ref[0] = jnp.sum(c1, axis=1, keepdims=True)


def _fold_kernel(s_ref, ss_ref, cnt_ref, w_ref, b_ref, scale_ref, shift_ref):
    s = s_ref[:, :, 0]
    ss = ss_ref[:, :, 0]
    cnt = cnt_ref[:, :, 0]
    safe = jnp.maximum(cnt, 1.0)
    mean = jnp.where(cnt > 0, s / safe, 0.0)
    sq = ss - 2.0 * mean * s + mean * mean * cnt
    var = jnp.where(cnt > 1, sq / safe, 1.0)
    b_count = s_ref.shape[0]
    fm = jnp.sum(mean, axis=0) / b_count
    fv = jnp.sum(var, axis=0) / b_count
    scale = w_ref[0] * jax.lax.rsqrt(fv + _EPS)
    shift = b_ref[0] - fm * scale
    scale_ref[...] = scale[:, None]
    shift_ref[...] = shift[:, None]


def _norm_kernel(scale_ref, shift_ref, x_ref, m_ref, o_ref):
    sc = scale_ref[...]
    sh = shift_ref[...]
    xb = x_ref[0]
    valid = m_ref[0, 0] > 0.5
    xn = xb * sc[:, :, None] + sh[:, :, None]
    o_ref[0] = jnp.where(valid[None], xn, xb)


def kernel(x, mask, weight, bias):
    B, C, H, W = x.shape

    s, ss, cnt = pl.pallas_call(
        _stats_kernel,
        grid=(B,),
        in_specs=[
            pl.BlockSpec((1, C, H, W), lambda b: (b, 0, 0, 0)),
            pl.BlockSpec((1, 1, H, W), lambda b: (b, 0, 0, 0)),
        ],
        out_specs=[
            pl.BlockSpec((1, C, 1), lambda b: (b, 0, 0)),
            pl.BlockSpec((1, C, 1), lambda b: (b, 0, 0)),
            pl.BlockSpec((1, 1, 1), lambda b: (b, 0, 0)),
        ],
        out_shape=[
            jax.ShapeDtypeStruct((B, C, 1), jnp.float32),
            jax.ShapeDtypeStruct((B, C, 1), jnp.float32),
            jax.ShapeDtypeStruct((B, 1, 1), jnp.float32),
        ],
        compiler_params=pltpu.CompilerParams(
            dimension_semantics=("parallel",),
            vmem_limit_bytes=52 * 1024 * 1024,
        ),
        name="masked_bn_stats",
    )(x, mask)

    w2 = weight.reshape(1, C)
    b2 = bias.reshape(1, C)

    scale, shift = pl.pallas_call(
        _fold_kernel,
        grid=(1,),
        in_specs=[
            pl.BlockSpec((B, C, 1), lambda i: (0, 0, 0)),
            pl.BlockSpec((B, C, 1), lambda i: (0, 0, 0)),
            pl.BlockSpec((B, 1, 1), lambda i: (0, 0, 0)),
            pl.BlockSpec((1, C), lambda i: (0, 0)),
            pl.BlockSpec((1, C), lambda i: (0, 0)),
        ],
        out_specs=[
            pl.BlockSpec((C, 1), lambda i: (0, 0)),
            pl.BlockSpec((C, 1), lambda i: (0, 0)),
        ],
        out_shape=[
            jax.ShapeDtypeStruct((C, 1), jnp.float32),
            jax.ShapeDtypeStruct((C, 1), jnp.float32),
        ],
        name="masked_bn_fold",
    )(s, ss, cnt, w2, b2)

    NJ = 4
    Hb = H // NJ
    out = pl.pallas_call(
        _norm_kernel,
        grid=(B, NJ),
        in_specs=[
            pl.BlockSpec((C, 1), lambda b, j: (0, 0)),
            pl.BlockSpec((C, 1), lambda b, j: (0, 0)),
            pl.BlockSpec((1, C, Hb, W), lambda b, j: (b, 0, j, 0)),
            pl.BlockSpec((1, 1, Hb, W), lambda b, j: (b, 0, j, 0)),
        ],
        out_specs=pl.BlockSpec((1, C, Hb, W), lambda b, j: (b, 0, j, 0)),
        out_shape=jax.ShapeDtypeStruct((B, C, H, W), jnp.float32),
        compiler_params=pltpu.CompilerParams(
            dimension_semantics=("parallel", "arbitrary"),
            vmem_limit_bytes=52 * 1024 * 1024,
        ),
        name="masked_bn_norm",
    )(scale, shift, x, mask)

    return out
